# baseline (speedup 1.0000x reference)
import functools

import jax
import jax.numpy as jnp
from jax import lax
from jax.experimental import pallas as pl
from jax.experimental.pallas import tpu as pltpu
from jax.experimental.pallas import tpu_sc as plsc

N, E, D = 10000, 320000, 128
TE = 2000
GRID_E = E // TE

NW = 32
EW = E // NW
KG = 400
KS = 160
NSTRIPE = 632


def _sc_gather(xB, xC, row, col):
    mesh = plsc.VectorSubcoreMesh(core_axis_name="c", subcore_axis_name="s")
    out = jax.ShapeDtypeStruct((E, D), jnp.float32)

    @functools.partial(
        pl.kernel, mesh=mesh, out_type=out,
        scratch_types=[
            pltpu.VMEM((KG,), jnp.int32),
            pltpu.VMEM((KG,), jnp.int32),
            pltpu.VMEM((KG, D), jnp.float32),
            pltpu.VMEM((KG, D), jnp.float32),
            pltpu.SemaphoreType.DMA,
            pltpu.SemaphoreType.DMA,
        ])
    def k(xB_hbm, xC_hbm, row_hbm, col_hbm, gsum_hbm,
          rowi_v, coli_v, bufB_v, bufC_v, sem1, sem2):
        wid = lax.axis_index("s") * 2 + lax.axis_index("c")
        base0 = wid * EW

        @pl.loop(0, EW, step=KG)
        def _(off):
            base = base0 + off
            pltpu.sync_copy(row_hbm.at[pl.ds(base, KG)], rowi_v)
            pltpu.sync_copy(col_hbm.at[pl.ds(base, KG)], coli_v)
            cpB = pltpu.async_copy(xB_hbm.at[rowi_v], bufB_v, sem1)
            cpC = pltpu.async_copy(xC_hbm.at[coli_v], bufC_v, sem2)
            cpB.wait()
            cpC.wait()

            @pl.loop(0, KG)
            def _(r):
                for j in range(8):
                    plsc.addupdate(bufB_v.at[r, pl.ds(j * 16, 16)],
                                   bufC_v[r, pl.ds(j * 16, 16)])

            pltpu.sync_copy(bufB_v, gsum_hbm.at[pl.ds(base, KG)])

    return k(xB, xC, row, col)


def _sc_scatter(sigma, xv, row, col, zeros_nd):
    mesh = plsc.VectorSubcoreMesh(core_axis_name="c", subcore_axis_name="s")
    out = jax.ShapeDtypeStruct((2, N, D), jnp.float32)
    NCHUNK = E // KS

    @functools.partial(
        pl.kernel, mesh=mesh, out_type=out,
        scratch_types=[
            pltpu.VMEM((KS,), jnp.int32),
            pltpu.VMEM((KS,), jnp.int32),
            pltpu.VMEM((KS, D), jnp.float32),
            pltpu.VMEM((KS, D), jnp.float32),
            pltpu.VMEM_SHARED((N, D), jnp.float32),
            pltpu.SemaphoreType.DMA,
            pltpu.SemaphoreType.DMA,
        ])
    def k(sig_hbm, xv_hbm, row_hbm, col_hbm, zero_hbm, agg_hbm,
          rowi_v, coli_v, sig_v, xvg_v, acc_sh, sem1, sem2):
        cid = lax.axis_index("c")
        sid = lax.axis_index("s")
        wid = sid * 2 + cid

        @pl.when(sid == 0)
        def _():
            pltpu.sync_copy(zero_hbm, acc_sh)

        plsc.subcore_barrier()

        @pl.loop(wid, NCHUNK, step=NW)
        def _(chunk):
            base = chunk * KS
            pltpu.sync_copy(col_hbm.at[pl.ds(base, KS)], coli_v)
            cpS = pltpu.async_copy(sig_hbm.at[pl.ds(base, KS)], sig_v, sem1)
            cpV = pltpu.async_copy(xv_hbm.at[coli_v], xvg_v, sem2)
            pltpu.sync_copy(row_hbm.at[pl.ds(base, KS)], rowi_v)
            cpS.wait()
            cpV.wait()

            @pl.loop(0, KS)
            def _(r):
                for j in range(8):
                    sl = (r, pl.ds(j * 16, 16))
                    sig_v[sl] = sig_v[sl] * xvg_v[sl]

            pltpu.sync_copy(sig_v, acc_sh.at[rowi_v], add=True)

        plsc.subcore_barrier()

        @pl.when(sid < 15)
        def _():
            pltpu.sync_copy(acc_sh.at[pl.ds(sid * NSTRIPE, NSTRIPE)],
                            agg_hbm.at[cid].at[pl.ds(sid * NSTRIPE, NSTRIPE)])

        @pl.when(sid == 15)
        def _():
            pltpu.sync_copy(acc_sh.at[pl.ds(15 * NSTRIPE, N - 15 * NSTRIPE)],
                            agg_hbm.at[cid].at[pl.ds(15 * NSTRIPE, N - 15 * NSTRIPE)])

    return k(sigma, xv, row, col, zeros_nd)


def _tables_body(x_ref, wu_ref, bu_ref, wv_ref, bv_ref, wb_ref, bb_ref,
                 wc_ref, bc_ref, xu_ref, xv_ref, xb_ref, xc_ref):
    xx = x_ref[...]
    xu_ref[...] = jnp.dot(xx, wu_ref[...], preferred_element_type=jnp.float32) + bu_ref[...]
    xv_ref[...] = jnp.dot(xx, wv_ref[...], preferred_element_type=jnp.float32) + bv_ref[...]
    xb_ref[...] = jnp.dot(xx, wb_ref[...], preferred_element_type=jnp.float32) + bb_ref[...]
    xc_ref[...] = jnp.dot(xx, wc_ref[...], preferred_element_type=jnp.float32) + bc_ref[...]


def _tables(x, WuT, bu, WvT, bv, WBT, bB, WCT, bC):
    out = jax.ShapeDtypeStruct((N, D), jnp.float32)
    return pl.pallas_call(
        _tables_body,
        out_shape=(out, out, out, out),
    )(x, WuT, bu, WvT, bv, WBT, bB, WCT, bC)


def _stats_body(ea_ref, gsum_ref, wat_ref, ba_ref, s1_ref, s2_ref, acc1, acc2):
    i = pl.program_id(0)

    @pl.when(i == 0)
    def _():
        acc1[...] = jnp.zeros_like(acc1)
        acc2[...] = jnp.zeros_like(acc2)

    ein = (jnp.dot(ea_ref[...], wat_ref[...], preferred_element_type=jnp.float32)
           + ba_ref[...] + gsum_ref[...])
    e3 = ein.reshape(TE // 8, 8, D)
    acc1[...] += jnp.sum(e3, axis=0)
    acc2[...] += jnp.sum(e3 * e3, axis=0)

    @pl.when(i == GRID_E - 1)
    def _():
        s1_ref[...] = acc1[...]
        s2_ref[...] = acc2[...]


def _edge_stats(edge_attr, gsum, WAT, bA):
    s = jax.ShapeDtypeStruct((8, D), jnp.float32)
    return pl.pallas_call(
        _stats_body,
        grid=(GRID_E,),
        in_specs=[
            pl.BlockSpec((TE, D), lambda i: (i, 0)),
            pl.BlockSpec((TE, D), lambda i: (i, 0)),
            pl.BlockSpec((D, D), lambda i: (0, 0)),
            pl.BlockSpec((1, D), lambda i: (0, 0)),
        ],
        out_specs=(pl.BlockSpec((8, D), lambda i: (0, 0)),
                   pl.BlockSpec((8, D), lambda i: (0, 0))),
        out_shape=(s, s),
        scratch_shapes=[pltpu.VMEM((8, D), jnp.float32),
                        pltpu.VMEM((8, D), jnp.float32)],
    )(edge_attr, gsum, WAT, bA)


def _apply_body(ea_ref, gsum_ref, wat_ref, ba_ref, s1_ref, s2_ref,
                ge_ref, be_ref, eout_ref, sig_ref):
    s1 = jnp.sum(s1_ref[...], axis=0, keepdims=True)
    s2 = jnp.sum(s2_ref[...], axis=0, keepdims=True)
    mean = s1 / E
    var = s2 / E - mean * mean
    rstd = jax.lax.rsqrt(var + 1e-5)
    ea = ea_ref[...]
    ein = (jnp.dot(ea, wat_ref[...], preferred_element_type=jnp.float32)
           + ba_ref[...] + gsum_ref[...])
    tmp = jnp.maximum(ge_ref[...] * (ein - mean) * rstd + be_ref[...], 0.0)
    eout = ea + tmp
    eout_ref[...] = eout
    sig_ref[...] = jax.nn.sigmoid(eout)


def _edge_apply(edge_attr, gsum, WAT, bA, s1, s2, gamma_e, beta_e):
    out = jax.ShapeDtypeStruct((E, D), jnp.float32)
    return pl.pallas_call(
        _apply_body,
        grid=(GRID_E,),
        in_specs=[
            pl.BlockSpec((TE, D), lambda i: (i, 0)),
            pl.BlockSpec((TE, D), lambda i: (i, 0)),
            pl.BlockSpec((D, D), lambda i: (0, 0)),
            pl.BlockSpec((1, D), lambda i: (0, 0)),
            pl.BlockSpec((8, D), lambda i: (0, 0)),
            pl.BlockSpec((8, D), lambda i: (0, 0)),
            pl.BlockSpec((1, D), lambda i: (0, 0)),
            pl.BlockSpec((1, D), lambda i: (0, 0)),
        ],
        out_specs=(pl.BlockSpec((TE, D), lambda i: (i, 0)),
                   pl.BlockSpec((TE, D), lambda i: (i, 0))),
        out_shape=(out, out),
    )(edge_attr, gsum, WAT, bA, s1, s2, gamma_e, beta_e)


def _node_body(x_ref, xu_ref, agg_ref, gn_ref, bn_ref, xo_ref):
    node_in = xu_ref[...] + agg_ref[0] + agg_ref[1]
    mean = jnp.mean(node_in, axis=0, keepdims=True)
    var = jnp.mean(node_in * node_in, axis=0, keepdims=True) - mean * mean
    rstd = jax.lax.rsqrt(var + 1e-5)
    tmp = jnp.maximum(gn_ref[...] * (node_in - mean) * rstd + bn_ref[...], 0.0)
    xo_ref[...] = x_ref[...] + tmp


def _node_final(x, xu, agg, gamma_n, beta_n):
    return pl.pallas_call(
        _node_body,
        out_shape=jax.ShapeDtypeStruct((N, D), jnp.float32),
    )(x, xu, agg, gamma_n, beta_n)


def kernel(x, edge_index, edge_attr, Wu, bu, Wv, bv, WA, bA, WB, bB, WC, bC,
           gamma_node, beta_node, gamma_edge, beta_edge):
    row = edge_index[0]
    col = edge_index[1]
    bu2 = bu.reshape(1, D)
    bv2 = bv.reshape(1, D)
    bA2 = bA.reshape(1, D)
    bB2 = bB.reshape(1, D)
    bC2 = bC.reshape(1, D)
    ge2 = gamma_edge.reshape(1, D)
    be2 = beta_edge.reshape(1, D)
    gn2 = gamma_node.reshape(1, D)
    bn2 = beta_node.reshape(1, D)

    xu, xv, xB, xC = _tables(x, Wu.T, bu2, Wv.T, bv2, WB.T, bB2, WC.T, bC2)

    gsum = _sc_gather(xB, xC, row, col)

    s1, s2 = _edge_stats(edge_attr, gsum, WA.T, bA2)
    eout, sigma = _edge_apply(edge_attr, gsum, WA.T, bA2, s1, s2, ge2, be2)

    zeros_nd = jnp.zeros((N, D), jnp.float32)
    agg = _sc_scatter(sigma, xv, row, col, zeros_nd)

    x_out = _node_final(x, xu, agg, gn2, bn2)
    return (x_out, eout)


# speedup vs baseline: 1.0856x; 1.0856x over previous
import functools

import jax
import jax.numpy as jnp
from jax import lax
from jax.experimental import pallas as pl
from jax.experimental.pallas import tpu as pltpu
from jax.experimental.pallas import tpu_sc as plsc

N, E, D = 10000, 320000, 128
EH = E // 2
TE = 2000
GRID_H = EH // TE

NW = 32
EWH = EH // NW
KG = 200
KS = 160
NCHUNK_S = EH // KS
NSTRIPE = 632


def _sc_gather(xB, xC, row_h, col_h):
    mesh = plsc.VectorSubcoreMesh(core_axis_name="c", subcore_axis_name="s")
    out = jax.ShapeDtypeStruct((EH, D), jnp.float32)

    @functools.partial(
        pl.kernel, mesh=mesh, out_type=out,
        scratch_types=[
            pltpu.VMEM((KG,), jnp.int32),
            pltpu.VMEM((KG,), jnp.int32),
            pltpu.VMEM((KG, D), jnp.float32),
            pltpu.VMEM((KG, D), jnp.float32),
            pltpu.SemaphoreType.DMA,
            pltpu.SemaphoreType.DMA,
        ])
    def k(xB_hbm, xC_hbm, row_hbm, col_hbm, gsum_hbm,
          rowi_v, coli_v, bufB_v, bufC_v, sem1, sem2):
        wid = lax.axis_index("s") * 2 + lax.axis_index("c")
        base0 = wid * EWH

        @pl.loop(0, EWH, step=KG)
        def _(off):
            base = base0 + off
            pltpu.sync_copy(row_hbm.at[pl.ds(base, KG)], rowi_v)
            pltpu.sync_copy(col_hbm.at[pl.ds(base, KG)], coli_v)
            cpB = pltpu.async_copy(xB_hbm.at[rowi_v], bufB_v, sem1)
            cpC = pltpu.async_copy(xC_hbm.at[coli_v], bufC_v, sem2)
            cpB.wait()
            cpC.wait()

            @pl.loop(0, KG)
            def _(r):
                for j in range(8):
                    plsc.addupdate(bufB_v.at[r, pl.ds(j * 16, 16)],
                                   bufC_v[r, pl.ds(j * 16, 16)])

            pltpu.sync_copy(bufB_v, gsum_hbm.at[pl.ds(base, KG)])

    return k(xB, xC, row_h, col_h)


def _sc_scatter(sigma_h, xv, row_h, col_h, zeros_nd):
    mesh = plsc.VectorSubcoreMesh(core_axis_name="c", subcore_axis_name="s")
    out = jax.ShapeDtypeStruct((2, N, D), jnp.float32)

    @functools.partial(
        pl.kernel, mesh=mesh, out_type=out,
        scratch_types=[
            pltpu.VMEM((KS,), jnp.int32),
            pltpu.VMEM((KS,), jnp.int32),
            pltpu.VMEM((KS, D), jnp.float32),
            pltpu.VMEM((KS, D), jnp.float32),
            pltpu.VMEM_SHARED((N, D), jnp.float32),
            pltpu.SemaphoreType.DMA,
            pltpu.SemaphoreType.DMA,
        ])
    def k(sig_hbm, xv_hbm, row_hbm, col_hbm, zero_hbm, agg_hbm,
          rowi_v, coli_v, sig_v, xvg_v, acc_sh, sem1, sem2):
        cid = lax.axis_index("c")
        sid = lax.axis_index("s")
        wid = sid * 2 + cid

        @pl.when(sid == 0)
        def _():
            pltpu.sync_copy(zero_hbm, acc_sh)

        plsc.subcore_barrier()

        @pl.loop(wid, NCHUNK_S, step=NW)
        def _(chunk):
            base = chunk * KS
            pltpu.sync_copy(col_hbm.at[pl.ds(base, KS)], coli_v)
            cpS = pltpu.async_copy(sig_hbm.at[pl.ds(base, KS)], sig_v, sem1)
            cpV = pltpu.async_copy(xv_hbm.at[coli_v], xvg_v, sem2)
            pltpu.sync_copy(row_hbm.at[pl.ds(base, KS)], rowi_v)
            cpS.wait()
            cpV.wait()

            @pl.loop(0, KS)
            def _(r):
                for j in range(8):
                    sl = (r, pl.ds(j * 16, 16))
                    sig_v[sl] = sig_v[sl] * xvg_v[sl]

            pltpu.sync_copy(sig_v, acc_sh.at[rowi_v], add=True)

        plsc.subcore_barrier()

        @pl.when(sid < 15)
        def _():
            pltpu.sync_copy(acc_sh.at[pl.ds(sid * NSTRIPE, NSTRIPE)],
                            agg_hbm.at[cid].at[pl.ds(sid * NSTRIPE, NSTRIPE)])

        @pl.when(sid == 15)
        def _():
            pltpu.sync_copy(acc_sh.at[pl.ds(15 * NSTRIPE, N - 15 * NSTRIPE)],
                            agg_hbm.at[cid].at[pl.ds(15 * NSTRIPE, N - 15 * NSTRIPE)])

    return k(sigma_h, xv, row_h, col_h, zeros_nd)


def _tables_body(x_ref, wu_ref, bu_ref, wv_ref, bv_ref, wb_ref, bb_ref,
                 wc_ref, bc_ref, xu_ref, xv_ref, xb_ref, xc_ref):
    xx = x_ref[...]
    xu_ref[...] = jnp.dot(xx, wu_ref[...], preferred_element_type=jnp.float32) + bu_ref[...]
    xv_ref[...] = jnp.dot(xx, wv_ref[...], preferred_element_type=jnp.float32) + bv_ref[...]
    xb_ref[...] = jnp.dot(xx, wb_ref[...], preferred_element_type=jnp.float32) + bb_ref[...]
    xc_ref[...] = jnp.dot(xx, wc_ref[...], preferred_element_type=jnp.float32) + bc_ref[...]


def _tables(x, WuT, bu, WvT, bv, WBT, bB, WCT, bC):
    out = jax.ShapeDtypeStruct((N, D), jnp.float32)
    return pl.pallas_call(
        _tables_body,
        out_shape=(out, out, out, out),
    )(x, WuT, bu, WvT, bv, WBT, bB, WCT, bC)


def _stats_body(ea_ref, gsum_ref, wat_ref, ba_ref, s1_ref, s2_ref, acc1, acc2):
    i = pl.program_id(0)

    @pl.when(i == 0)
    def _():
        acc1[...] = jnp.zeros_like(acc1)
        acc2[...] = jnp.zeros_like(acc2)

    ein = (jnp.dot(ea_ref[...], wat_ref[...], preferred_element_type=jnp.float32)
           + ba_ref[...] + gsum_ref[...])
    e3 = ein.reshape(TE // 8, 8, D)
    acc1[...] += jnp.sum(e3, axis=0)
    acc2[...] += jnp.sum(e3 * e3, axis=0)

    @pl.when(i == GRID_H - 1)
    def _():
        s1_ref[...] = acc1[...]
        s2_ref[...] = acc2[...]


def _edge_stats(edge_attr, gsum_h, WAT, bA, off):
    s = jax.ShapeDtypeStruct((8, D), jnp.float32)
    noff = off // TE
    return pl.pallas_call(
        _stats_body,
        grid=(GRID_H,),
        in_specs=[
            pl.BlockSpec((TE, D), lambda i: (i + noff, 0)),
            pl.BlockSpec((TE, D), lambda i: (i, 0)),
            pl.BlockSpec((D, D), lambda i: (0, 0)),
            pl.BlockSpec((1, D), lambda i: (0, 0)),
        ],
        out_specs=(pl.BlockSpec((8, D), lambda i: (0, 0)),
                   pl.BlockSpec((8, D), lambda i: (0, 0))),
        out_shape=(s, s),
        scratch_shapes=[pltpu.VMEM((8, D), jnp.float32),
                        pltpu.VMEM((8, D), jnp.float32)],
    )(edge_attr, gsum_h, WAT, bA)


def _apply_common(ea_ref, gsum_ref, wat_ref, ba_ref, s1a_ref, s1b_ref,
                  s2a_ref, s2b_ref, ge_ref, be_ref, eout_ref, sig_ref):
    s1 = jnp.sum(s1a_ref[...] + s1b_ref[...], axis=0, keepdims=True)
    s2 = jnp.sum(s2a_ref[...] + s2b_ref[...], axis=0, keepdims=True)
    mean = s1 / E
    var = s2 / E - mean * mean
    rstd = jax.lax.rsqrt(var + 1e-5)
    ea = ea_ref[...]
    ein = (jnp.dot(ea, wat_ref[...], preferred_element_type=jnp.float32)
           + ba_ref[...] + gsum_ref[...])
    tmp = jnp.maximum(ge_ref[...] * (ein - mean) * rstd + be_ref[...], 0.0)
    eout = ea + tmp
    eout_ref[...] = eout
    sig_ref[...] = jax.nn.sigmoid(eout)


def _apply_body_first(ea_ref, gsum_ref, wat_ref, ba_ref, s1a_ref, s1b_ref,
                      s2a_ref, s2b_ref, ge_ref, be_ref, eout_ref, sig_ref):
    _apply_common(ea_ref, gsum_ref, wat_ref, ba_ref, s1a_ref, s1b_ref,
                  s2a_ref, s2b_ref, ge_ref, be_ref, eout_ref, sig_ref)


def _apply_body_next(ea_ref, gsum_ref, wat_ref, ba_ref, s1a_ref, s1b_ref,
                     s2a_ref, s2b_ref, ge_ref, be_ref, eprev_ref,
                     eout_ref, sig_ref):
    del eprev_ref
    _apply_common(ea_ref, gsum_ref, wat_ref, ba_ref, s1a_ref, s1b_ref,
                  s2a_ref, s2b_ref, ge_ref, be_ref, eout_ref, sig_ref)


def _edge_apply(edge_attr, gsum_h, WAT, bA, s1a, s1b, s2a, s2b,
                gamma_e, beta_e, eout_prev, off):
    noff = off // TE
    out_full = jax.ShapeDtypeStruct((E, D), jnp.float32)
    out_half = jax.ShapeDtypeStruct((EH, D), jnp.float32)
    in_specs = [
        pl.BlockSpec((TE, D), lambda i: (i + noff, 0)),
        pl.BlockSpec((TE, D), lambda i: (i, 0)),
        pl.BlockSpec((D, D), lambda i: (0, 0)),
        pl.BlockSpec((1, D), lambda i: (0, 0)),
        pl.BlockSpec((8, D), lambda i: (0, 0)),
        pl.BlockSpec((8, D), lambda i: (0, 0)),
        pl.BlockSpec((8, D), lambda i: (0, 0)),
        pl.BlockSpec((8, D), lambda i: (0, 0)),
        pl.BlockSpec((1, D), lambda i: (0, 0)),
        pl.BlockSpec((1, D), lambda i: (0, 0)),
    ]
    args = [edge_attr, gsum_h, WAT, bA, s1a, s1b, s2a, s2b, gamma_e, beta_e]
    if eout_prev is None:
        body, aliases = _apply_body_first, {}
    else:
        in_specs.append(pl.BlockSpec((8, D), lambda i: (0, 0)))
        args.append(eout_prev)
        body, aliases = _apply_body_next, {10: 0}
    return pl.pallas_call(
        body,
        grid=(GRID_H,),
        in_specs=in_specs,
        out_specs=(pl.BlockSpec((TE, D), lambda i: (i + noff, 0)),
                   pl.BlockSpec((TE, D), lambda i: (i, 0))),
        out_shape=(out_full, out_half),
        input_output_aliases=aliases,
    )(*args)


def _node_body(x_ref, xu_ref, agga_ref, aggb_ref, gn_ref, bn_ref, xo_ref):
    node_in = (xu_ref[...] + agga_ref[0] + agga_ref[1]
               + aggb_ref[0] + aggb_ref[1])
    mean = jnp.mean(node_in, axis=0, keepdims=True)
    var = jnp.mean(node_in * node_in, axis=0, keepdims=True) - mean * mean
    rstd = jax.lax.rsqrt(var + 1e-5)
    tmp = jnp.maximum(gn_ref[...] * (node_in - mean) * rstd + bn_ref[...], 0.0)
    xo_ref[...] = x_ref[...] + tmp


def _node_final(x, xu, agg_a, agg_b, gamma_n, beta_n):
    return pl.pallas_call(
        _node_body,
        out_shape=jax.ShapeDtypeStruct((N, D), jnp.float32),
    )(x, xu, agg_a, agg_b, gamma_n, beta_n)


def kernel(x, edge_index, edge_attr, Wu, bu, Wv, bv, WA, bA, WB, bB, WC, bC,
           gamma_node, beta_node, gamma_edge, beta_edge):
    row = edge_index[0]
    col = edge_index[1]
    row_lo, row_hi = row[:EH], row[EH:]
    col_lo, col_hi = col[:EH], col[EH:]
    bu2 = bu.reshape(1, D)
    bv2 = bv.reshape(1, D)
    bA2 = bA.reshape(1, D)
    bB2 = bB.reshape(1, D)
    bC2 = bC.reshape(1, D)
    ge2 = gamma_edge.reshape(1, D)
    be2 = beta_edge.reshape(1, D)
    gn2 = gamma_node.reshape(1, D)
    bn2 = beta_node.reshape(1, D)

    xu, xv, xB, xC = _tables(x, Wu.T, bu2, Wv.T, bv2, WB.T, bB2, WC.T, bC2)

    gsum_lo = _sc_gather(xB, xC, row_lo, col_lo)
    gsum_hi = _sc_gather(xB, xC, row_hi, col_hi)

    WAT = WA.T
    s1a, s2a = _edge_stats(edge_attr, gsum_lo, WAT, bA2, 0)
    s1b, s2b = _edge_stats(edge_attr, gsum_hi, WAT, bA2, EH)

    eout1, sig_lo = _edge_apply(edge_attr, gsum_lo, WAT, bA2,
                                s1a, s1b, s2a, s2b, ge2, be2, None, 0)
    eout, sig_hi = _edge_apply(edge_attr, gsum_hi, WAT, bA2,
                               s1a, s1b, s2a, s2b, ge2, be2, eout1, EH)

    zeros_nd = jnp.zeros((N, D), jnp.float32)
    agg_a = _sc_scatter(sig_lo, xv, row_lo, col_lo, zeros_nd)
    agg_b = _sc_scatter(sig_hi, xv, row_hi, col_hi, zeros_nd)

    x_out = _node_final(x, xu, agg_a, agg_b, gn2, bn2)
    return (x_out, eout)


# speedup vs baseline: 1.2188x; 1.1228x over previous
import functools

import jax
import jax.numpy as jnp
from jax import lax
from jax.experimental import pallas as pl
from jax.experimental.pallas import tpu as pltpu
from jax.experimental.pallas import tpu_sc as plsc

N, E, D = 10000, 320000, 128
EH = E // 2
TE = 2000
GRID_H = EH // TE

NW = 32
EWH = EH // NW
KG = 200
KS = 160
NCHUNK_S = EH // KS
NSTRIPE = 632


NCH_G = EWH // KG


def _sc_gather(xB, xC, row_h, col_h):
    mesh = plsc.VectorSubcoreMesh(core_axis_name="c", subcore_axis_name="s")
    out = jax.ShapeDtypeStruct((EH, D), jnp.float32)

    @functools.partial(
        pl.kernel, mesh=mesh, out_type=out,
        scratch_types=[
            pltpu.VMEM((KG,), jnp.int32),
            pltpu.VMEM((KG,), jnp.int32),
            pltpu.VMEM((KG, D), jnp.float32),
            pltpu.VMEM((KG, D), jnp.float32),
            pltpu.VMEM((KG,), jnp.int32),
            pltpu.VMEM((KG,), jnp.int32),
            pltpu.VMEM((KG, D), jnp.float32),
            pltpu.VMEM((KG, D), jnp.float32),
            pltpu.SemaphoreType.DMA,
            pltpu.SemaphoreType.DMA,
            pltpu.SemaphoreType.DMA,
            pltpu.SemaphoreType.DMA,
        ])
    def k(xB_hbm, xC_hbm, row_hbm, col_hbm, gsum_hbm,
          rowi0, coli0, bufB0, bufC0, rowi1, coli1, bufB1, bufC1,
          semB0, semC0, semB1, semC1):
        wid = lax.axis_index("s") * 2 + lax.axis_index("c")
        base0 = wid * EWH

        def start(off, rowi, coli, bufB, bufC, semB, semC):
            base = base0 + off
            pltpu.sync_copy(row_hbm.at[pl.ds(base, KG)], rowi)
            pltpu.sync_copy(col_hbm.at[pl.ds(base, KG)], coli)
            pltpu.async_copy(xB_hbm.at[rowi], bufB, semB)
            pltpu.async_copy(xC_hbm.at[coli], bufC, semC)

        def finish(off, rowi, coli, bufB, bufC, semB, semC):
            base = base0 + off
            pltpu.make_async_copy(xB_hbm.at[rowi], bufB, semB).wait()
            pltpu.make_async_copy(xC_hbm.at[coli], bufC, semC).wait()

            @pl.loop(0, KG)
            def _(r):
                for j in range(8):
                    plsc.addupdate(bufB.at[r, pl.ds(j * 16, 16)],
                                   bufC[r, pl.ds(j * 16, 16)])

            pltpu.sync_copy(bufB, gsum_hbm.at[pl.ds(base, KG)])

        set0 = (rowi0, coli0, bufB0, bufC0, semB0, semC0)
        set1 = (rowi1, coli1, bufB1, bufC1, semB1, semC1)

        start(0, *set0)

        @pl.loop(0, (NCH_G - 1) * KG, step=2 * KG)
        def _(off):
            start(off + KG, *set1)
            finish(off, *set0)

            @pl.when(off + 2 * KG < NCH_G * KG)
            def _():
                start(off + 2 * KG, *set0)

            finish(off + KG, *set1)

        finish((NCH_G - 1) * KG, *set0)

    return k(xB, xC, row_h, col_h)


def _sc_scatter(sigma_h, xv, row_h, col_h, zeros_nd):
    mesh = plsc.VectorSubcoreMesh(core_axis_name="c", subcore_axis_name="s")
    out = jax.ShapeDtypeStruct((2, N, D), jnp.float32)

    @functools.partial(
        pl.kernel, mesh=mesh, out_type=out,
        scratch_types=[
            pltpu.VMEM((KS,), jnp.int32),
            pltpu.VMEM((KS,), jnp.int32),
            pltpu.VMEM((KS, D), jnp.float32),
            pltpu.VMEM((KS, D), jnp.float32),
            pltpu.VMEM_SHARED((N, D), jnp.float32),
            pltpu.SemaphoreType.DMA,
            pltpu.SemaphoreType.DMA,
        ])
    def k(sig_hbm, xv_hbm, row_hbm, col_hbm, zero_hbm, agg_hbm,
          rowi_v, coli_v, sig_v, xvg_v, acc_sh, sem1, sem2):
        cid = lax.axis_index("c")
        sid = lax.axis_index("s")
        wid = sid * 2 + cid

        @pl.when(sid == 0)
        def _():
            pltpu.sync_copy(zero_hbm, acc_sh)

        plsc.subcore_barrier()

        @pl.loop(wid, NCHUNK_S, step=NW)
        def _(chunk):
            base = chunk * KS
            pltpu.sync_copy(col_hbm.at[pl.ds(base, KS)], coli_v)
            cpS = pltpu.async_copy(sig_hbm.at[pl.ds(base, KS)], sig_v, sem1)
            cpV = pltpu.async_copy(xv_hbm.at[coli_v], xvg_v, sem2)
            pltpu.sync_copy(row_hbm.at[pl.ds(base, KS)], rowi_v)
            cpS.wait()
            cpV.wait()

            @pl.loop(0, KS)
            def _(r):
                for j in range(8):
                    sl = (r, pl.ds(j * 16, 16))
                    sig_v[sl] = sig_v[sl] * xvg_v[sl]

            pltpu.sync_copy(sig_v, acc_sh.at[rowi_v], add=True)

        plsc.subcore_barrier()

        @pl.when(sid < 15)
        def _():
            pltpu.sync_copy(acc_sh.at[pl.ds(sid * NSTRIPE, NSTRIPE)],
                            agg_hbm.at[cid].at[pl.ds(sid * NSTRIPE, NSTRIPE)])

        @pl.when(sid == 15)
        def _():
            pltpu.sync_copy(acc_sh.at[pl.ds(15 * NSTRIPE, N - 15 * NSTRIPE)],
                            agg_hbm.at[cid].at[pl.ds(15 * NSTRIPE, N - 15 * NSTRIPE)])

    return k(sigma_h, xv, row_h, col_h, zeros_nd)


def _tables_body(x_ref, wu_ref, bu_ref, wv_ref, bv_ref, wb_ref, bb_ref,
                 wc_ref, bc_ref, xu_ref, xv_ref, xb_ref, xc_ref):
    xx = x_ref[...]
    xu_ref[...] = jnp.dot(xx, wu_ref[...], preferred_element_type=jnp.float32) + bu_ref[...]
    xv_ref[...] = jnp.dot(xx, wv_ref[...], preferred_element_type=jnp.float32) + bv_ref[...]
    xb_ref[...] = jnp.dot(xx, wb_ref[...], preferred_element_type=jnp.float32) + bb_ref[...]
    xc_ref[...] = jnp.dot(xx, wc_ref[...], preferred_element_type=jnp.float32) + bc_ref[...]


def _tables(x, WuT, bu, WvT, bv, WBT, bB, WCT, bC):
    out = jax.ShapeDtypeStruct((N, D), jnp.float32)
    return pl.pallas_call(
        _tables_body,
        out_shape=(out, out, out, out),
    )(x, WuT, bu, WvT, bv, WBT, bB, WCT, bC)


def _stats_body(ea_ref, gsum_ref, wat_ref, ba_ref, s1_ref, s2_ref, acc1, acc2):
    i = pl.program_id(0)

    @pl.when(i == 0)
    def _():
        acc1[...] = jnp.zeros_like(acc1)
        acc2[...] = jnp.zeros_like(acc2)

    ein = (jnp.dot(ea_ref[...], wat_ref[...], preferred_element_type=jnp.float32)
           + ba_ref[...] + gsum_ref[...])
    e3 = ein.reshape(TE // 8, 8, D)
    acc1[...] += jnp.sum(e3, axis=0)
    acc2[...] += jnp.sum(e3 * e3, axis=0)

    @pl.when(i == GRID_H - 1)
    def _():
        s1_ref[...] = acc1[...]
        s2_ref[...] = acc2[...]


def _edge_stats(edge_attr, gsum_h, WAT, bA, off):
    s = jax.ShapeDtypeStruct((8, D), jnp.float32)
    noff = off // TE
    return pl.pallas_call(
        _stats_body,
        grid=(GRID_H,),
        in_specs=[
            pl.BlockSpec((TE, D), lambda i: (i + noff, 0)),
            pl.BlockSpec((TE, D), lambda i: (i, 0)),
            pl.BlockSpec((D, D), lambda i: (0, 0)),
            pl.BlockSpec((1, D), lambda i: (0, 0)),
        ],
        out_specs=(pl.BlockSpec((8, D), lambda i: (0, 0)),
                   pl.BlockSpec((8, D), lambda i: (0, 0))),
        out_shape=(s, s),
        scratch_shapes=[pltpu.VMEM((8, D), jnp.float32),
                        pltpu.VMEM((8, D), jnp.float32)],
    )(edge_attr, gsum_h, WAT, bA)


def _apply_common(ea_ref, gsum_ref, wat_ref, ba_ref, s1a_ref, s1b_ref,
                  s2a_ref, s2b_ref, ge_ref, be_ref, eout_ref, sig_ref):
    s1 = jnp.sum(s1a_ref[...] + s1b_ref[...], axis=0, keepdims=True)
    s2 = jnp.sum(s2a_ref[...] + s2b_ref[...], axis=0, keepdims=True)
    mean = s1 / E
    var = s2 / E - mean * mean
    rstd = jax.lax.rsqrt(var + 1e-5)
    ea = ea_ref[...]
    ein = (jnp.dot(ea, wat_ref[...], preferred_element_type=jnp.float32)
           + ba_ref[...] + gsum_ref[...])
    tmp = jnp.maximum(ge_ref[...] * (ein - mean) * rstd + be_ref[...], 0.0)
    eout = ea + tmp
    eout_ref[...] = eout
    sig_ref[...] = jax.nn.sigmoid(eout)


def _apply_body_first(ea_ref, gsum_ref, wat_ref, ba_ref, s1a_ref, s1b_ref,
                      s2a_ref, s2b_ref, ge_ref, be_ref, eout_ref, sig_ref):
    _apply_common(ea_ref, gsum_ref, wat_ref, ba_ref, s1a_ref, s1b_ref,
                  s2a_ref, s2b_ref, ge_ref, be_ref, eout_ref, sig_ref)


def _apply_body_next(ea_ref, gsum_ref, wat_ref, ba_ref, s1a_ref, s1b_ref,
                     s2a_ref, s2b_ref, ge_ref, be_ref, eprev_ref,
                     eout_ref, sig_ref):
    del eprev_ref
    _apply_common(ea_ref, gsum_ref, wat_ref, ba_ref, s1a_ref, s1b_ref,
                  s2a_ref, s2b_ref, ge_ref, be_ref, eout_ref, sig_ref)


def _edge_apply(edge_attr, gsum_h, WAT, bA, s1a, s1b, s2a, s2b,
                gamma_e, beta_e, eout_prev, off):
    noff = off // TE
    out_full = jax.ShapeDtypeStruct((E, D), jnp.float32)
    out_half = jax.ShapeDtypeStruct((EH, D), jnp.float32)
    in_specs = [
        pl.BlockSpec((TE, D), lambda i: (i + noff, 0)),
        pl.BlockSpec((TE, D), lambda i: (i, 0)),
        pl.BlockSpec((D, D), lambda i: (0, 0)),
        pl.BlockSpec((1, D), lambda i: (0, 0)),
        pl.BlockSpec((8, D), lambda i: (0, 0)),
        pl.BlockSpec((8, D), lambda i: (0, 0)),
        pl.BlockSpec((8, D), lambda i: (0, 0)),
        pl.BlockSpec((8, D), lambda i: (0, 0)),
        pl.BlockSpec((1, D), lambda i: (0, 0)),
        pl.BlockSpec((1, D), lambda i: (0, 0)),
    ]
    args = [edge_attr, gsum_h, WAT, bA, s1a, s1b, s2a, s2b, gamma_e, beta_e]
    if eout_prev is None:
        body, aliases = _apply_body_first, {}
    else:
        in_specs.append(pl.BlockSpec((8, D), lambda i: (0, 0)))
        args.append(eout_prev)
        body, aliases = _apply_body_next, {10: 0}
    return pl.pallas_call(
        body,
        grid=(GRID_H,),
        in_specs=in_specs,
        out_specs=(pl.BlockSpec((TE, D), lambda i: (i + noff, 0)),
                   pl.BlockSpec((TE, D), lambda i: (i, 0))),
        out_shape=(out_full, out_half),
        input_output_aliases=aliases,
    )(*args)


def _node_body(x_ref, xu_ref, agga_ref, aggb_ref, gn_ref, bn_ref, xo_ref):
    node_in = (xu_ref[...] + agga_ref[0] + agga_ref[1]
               + aggb_ref[0] + aggb_ref[1])
    mean = jnp.mean(node_in, axis=0, keepdims=True)
    var = jnp.mean(node_in * node_in, axis=0, keepdims=True) - mean * mean
    rstd = jax.lax.rsqrt(var + 1e-5)
    tmp = jnp.maximum(gn_ref[...] * (node_in - mean) * rstd + bn_ref[...], 0.0)
    xo_ref[...] = x_ref[...] + tmp


def _node_final(x, xu, agg_a, agg_b, gamma_n, beta_n):
    return pl.pallas_call(
        _node_body,
        out_shape=jax.ShapeDtypeStruct((N, D), jnp.float32),
    )(x, xu, agg_a, agg_b, gamma_n, beta_n)


def kernel(x, edge_index, edge_attr, Wu, bu, Wv, bv, WA, bA, WB, bB, WC, bC,
           gamma_node, beta_node, gamma_edge, beta_edge):
    row = edge_index[0]
    col = edge_index[1]
    row_lo, row_hi = row[:EH], row[EH:]
    col_lo, col_hi = col[:EH], col[EH:]
    bu2 = bu.reshape(1, D)
    bv2 = bv.reshape(1, D)
    bA2 = bA.reshape(1, D)
    bB2 = bB.reshape(1, D)
    bC2 = bC.reshape(1, D)
    ge2 = gamma_edge.reshape(1, D)
    be2 = beta_edge.reshape(1, D)
    gn2 = gamma_node.reshape(1, D)
    bn2 = beta_node.reshape(1, D)

    xu, xv, xB, xC = _tables(x, Wu.T, bu2, Wv.T, bv2, WB.T, bB2, WC.T, bC2)

    gsum_lo = _sc_gather(xB, xC, row_lo, col_lo)
    gsum_hi = _sc_gather(xB, xC, row_hi, col_hi)

    WAT = WA.T
    s1a, s2a = _edge_stats(edge_attr, gsum_lo, WAT, bA2, 0)
    s1b, s2b = _edge_stats(edge_attr, gsum_hi, WAT, bA2, EH)

    eout1, sig_lo = _edge_apply(edge_attr, gsum_lo, WAT, bA2,
                                s1a, s1b, s2a, s2b, ge2, be2, None, 0)
    eout, sig_hi = _edge_apply(edge_attr, gsum_hi, WAT, bA2,
                               s1a, s1b, s2a, s2b, ge2, be2, eout1, EH)

    zeros_nd = jnp.zeros((N, D), jnp.float32)
    agg_a = _sc_scatter(sig_lo, xv, row_lo, col_lo, zeros_nd)
    agg_b = _sc_scatter(sig_hi, xv, row_hi, col_hi, zeros_nd)

    x_out = _node_final(x, xu, agg_a, agg_b, gn2, bn2)
    return (x_out, eout)
